# baseline (speedup 1.0000x reference)
import math

import jax
import jax.numpy as jnp
from jax.experimental import pallas as pl
from jax.experimental.pallas import tpu as pltpu

N = 100000
D = 128
H = 128
NUM_GRAPHS = 512
BN_EPS = 1e-5
BLK = 4000
NBLK = N // BLK
WIN = 64
_BN_C = float(1.0 / math.sqrt(1.0 + BN_EPS))


def _fused_kernel(nf_ref, sm_ref, gid_ref, wrep_ref, ba_ref, ones_ref,
                  W0_ref, b0_ref, a0_ref, g0_ref, be0_ref,
                  W1_ref, b1_ref, a1_ref, g1_ref, be1_ref,
                  W2_ref, b2_ref, a2_ref, g2_ref, be2_ref,
                  Wp_ref, bp_ref,
                  w_out_ref, out_ref, acc_ref):
    i = pl.program_id(0)
    nf = nf_ref[...]
    logit = jnp.dot(nf, wrep_ref[...], precision=jax.lax.Precision.HIGHEST,
                    preferred_element_type=jnp.float32)
    sm_full = jnp.dot(sm_ref[...], ones_ref[...],
                      preferred_element_type=jnp.float32)
    w_full = jax.nn.sigmoid(logit + ba_ref[0, 0]) * sm_full
    w_out_ref[...] = w_full[:, :1]
    x = nf * w_full
    gid = gid_ref[0, 0, :]

    @pl.when(i == 0)
    def _init():
        acc_ref[...] = jnp.zeros_like(acc_ref)

    first = gid_ref[0, 0, 0]
    last = gid_ref[0, 0, BLK - 1]
    row_iota = jax.lax.broadcasted_iota(jnp.int32, (WIN, BLK), 0)

    def _cond(carry):
        return carry <= last

    def _body(carry):
        base = jnp.minimum((carry // 8) * 8, NUM_GRAPHS - WIN)
        sel = (gid >= carry) & (gid < base + WIN)
        onehot = jnp.where(sel[None, :] & (row_iota == (gid - base)[None, :]),
                           1.0, 0.0)
        acc_ref[pl.ds(base, WIN), :] += jnp.dot(
            onehot, x, preferred_element_type=jnp.float32)
        return base + WIN

    jax.lax.while_loop(_cond, _body, first)

    @pl.when(i == NBLK - 1)
    def _head():
        h = acc_ref[...]
        for W, b, a, g, be in ((W0_ref, b0_ref, a0_ref, g0_ref, be0_ref),
                               (W1_ref, b1_ref, a1_ref, g1_ref, be1_ref),
                               (W2_ref, b2_ref, a2_ref, g2_ref, be2_ref)):
            h = jnp.dot(h, W[...], precision=jax.lax.Precision.HIGHEST,
                        preferred_element_type=jnp.float32) + b[...]
            h = jnp.where(h >= 0, h, a[0, 0] * h)
            h = g[...] * (h * _BN_C) + be[...]
        out_ref[...] = jnp.dot(h, Wp_ref[...], precision=jax.lax.Precision.HIGHEST,
                               preferred_element_type=jnp.float32) + bp_ref[...]


@jax.jit
def kernel(node_feats, edge_feats, smask, graph_ids, w_atom, b_atom,
           W0, b0, a0, g0, be0, W1, b1, a1, g1, be1, W2, b2, a2, g2, be2,
           Wp, bp):
    del edge_feats
    sm = smask.reshape(N, 1)
    gid3 = graph_ids.reshape(NBLK, 1, BLK)
    w_rep = jnp.tile(w_atom, (1, D))
    ones_row = jnp.ones((1, D), jnp.float32)

    full = lambda *shape: pl.BlockSpec(shape, lambda i: (0,) * len(shape))
    in_specs = [
        pl.BlockSpec((BLK, D), lambda i: (i, 0)),
        pl.BlockSpec((BLK, 1), lambda i: (i, 0)),
        pl.BlockSpec((1, 1, BLK), lambda i: (i, 0, 0)),
        full(D, D), full(1, 1), full(1, D),
    ]
    for _ in range(3):
        in_specs += [full(D, H), full(1, H), full(1, 1), full(1, H), full(1, H)]
    in_specs += [full(H, 1), full(1, 1)]

    w_out, out = pl.pallas_call(
        _fused_kernel,
        grid=(NBLK,),
        in_specs=in_specs,
        out_specs=[
            pl.BlockSpec((BLK, 1), lambda i: (i, 0)),
            pl.BlockSpec((NUM_GRAPHS, 1), lambda i: (0, 0)),
        ],
        out_shape=[
            jax.ShapeDtypeStruct((N, 1), jnp.float32),
            jax.ShapeDtypeStruct((NUM_GRAPHS, 1), jnp.float32),
        ],
        scratch_shapes=[pltpu.VMEM((NUM_GRAPHS, H), jnp.float32)],
    )(node_feats, sm, gid3, w_rep, b_atom.reshape(1, 1), ones_row,
      W0, b0.reshape(1, H), a0.reshape(1, 1), g0.reshape(1, H), be0.reshape(1, H),
      W1, b1.reshape(1, H), a1.reshape(1, 1), g1.reshape(1, H), be1.reshape(1, H),
      W2, b2.reshape(1, H), a2.reshape(1, 1), g2.reshape(1, H), be2.reshape(1, H),
      Wp, bp.reshape(1, 1))
    return out, w_out


# speedup vs baseline: 1.0418x; 1.0418x over previous
import math

import jax
import jax.numpy as jnp
from jax.experimental import pallas as pl
from jax.experimental.pallas import tpu as pltpu

N = 100000
D = 128
H = 128
NUM_GRAPHS = 512
BN_EPS = 1e-5
BLK = 4000
NBLK = N // BLK
WIN = 64
_BN_C = float(1.0 / math.sqrt(1.0 + BN_EPS))


def _fused_kernel(nf_ref, sm_ref, gid_ref, wrep_ref, ba_ref, ones_ref,
                  W0_ref, b0_ref, a0_ref, g0_ref, be0_ref,
                  W1_ref, b1_ref, a1_ref, g1_ref, be1_ref,
                  W2_ref, b2_ref, a2_ref, g2_ref, be2_ref,
                  Wp_ref, bp_ref,
                  w_out_ref, out_ref, acc_ref):
    i = pl.program_id(0)
    nf = nf_ref[...]
    logit = jnp.dot(nf, wrep_ref[...],
                    preferred_element_type=jnp.float32)
    sm_full = jnp.dot(sm_ref[...], ones_ref[...],
                      preferred_element_type=jnp.float32)
    w_full = jax.nn.sigmoid(logit + ba_ref[0, 0]) * sm_full
    w_out_ref[...] = w_full[:, :1]
    x = nf * w_full
    x_hi = x.astype(jnp.bfloat16)
    x_lo = (x - x_hi.astype(jnp.float32)).astype(jnp.bfloat16)
    gid = gid_ref[0, 0, :]

    @pl.when(i == 0)
    def _init():
        acc_ref[...] = jnp.zeros_like(acc_ref)

    first = gid_ref[0, 0, 0]
    last = gid_ref[0, 0, BLK - 1]
    row_iota = jax.lax.broadcasted_iota(jnp.int32, (WIN, BLK), 0)

    def _cond(carry):
        return carry <= last

    def _body(carry):
        base = jnp.minimum((carry // 8) * 8, NUM_GRAPHS - WIN)
        sel = (gid >= carry) & (gid < base + WIN)
        onehot = jnp.where(sel[None, :] & (row_iota == (gid - base)[None, :]),
                           1.0, 0.0).astype(jnp.bfloat16)
        acc_ref[pl.ds(base, WIN), :] += (
            jnp.dot(onehot, x_hi, preferred_element_type=jnp.float32)
            + jnp.dot(onehot, x_lo, preferred_element_type=jnp.float32))
        return base + WIN

    jax.lax.while_loop(_cond, _body, first)

    @pl.when(i == NBLK - 1)
    def _head():
        h = acc_ref[...]
        for W, b, a, g, be in ((W0_ref, b0_ref, a0_ref, g0_ref, be0_ref),
                               (W1_ref, b1_ref, a1_ref, g1_ref, be1_ref),
                               (W2_ref, b2_ref, a2_ref, g2_ref, be2_ref)):
            h = jnp.dot(h, W[...], precision=jax.lax.Precision.HIGHEST,
                        preferred_element_type=jnp.float32) + b[...]
            h = jnp.where(h >= 0, h, a[0, 0] * h)
            h = g[...] * (h * _BN_C) + be[...]
        out_ref[...] = jnp.dot(h, Wp_ref[...], precision=jax.lax.Precision.HIGHEST,
                               preferred_element_type=jnp.float32) + bp_ref[...]


@jax.jit
def kernel(node_feats, edge_feats, smask, graph_ids, w_atom, b_atom,
           W0, b0, a0, g0, be0, W1, b1, a1, g1, be1, W2, b2, a2, g2, be2,
           Wp, bp):
    del edge_feats
    sm = smask.reshape(N, 1)
    gid3 = graph_ids.reshape(NBLK, 1, BLK)
    w_rep = jnp.tile(w_atom, (1, D))
    ones_row = jnp.ones((1, D), jnp.float32)

    full = lambda *shape: pl.BlockSpec(shape, lambda i: (0,) * len(shape))
    in_specs = [
        pl.BlockSpec((BLK, D), lambda i: (i, 0)),
        pl.BlockSpec((BLK, 1), lambda i: (i, 0)),
        pl.BlockSpec((1, 1, BLK), lambda i: (i, 0, 0)),
        full(D, D), full(1, 1), full(1, D),
    ]
    for _ in range(3):
        in_specs += [full(D, H), full(1, H), full(1, 1), full(1, H), full(1, H)]
    in_specs += [full(H, 1), full(1, 1)]

    w_out, out = pl.pallas_call(
        _fused_kernel,
        grid=(NBLK,),
        in_specs=in_specs,
        out_specs=[
            pl.BlockSpec((BLK, 1), lambda i: (i, 0)),
            pl.BlockSpec((NUM_GRAPHS, 1), lambda i: (0, 0)),
        ],
        out_shape=[
            jax.ShapeDtypeStruct((N, 1), jnp.float32),
            jax.ShapeDtypeStruct((NUM_GRAPHS, 1), jnp.float32),
        ],
        scratch_shapes=[pltpu.VMEM((NUM_GRAPHS, H), jnp.float32)],
    )(node_feats, sm, gid3, w_rep, b_atom.reshape(1, 1), ones_row,
      W0, b0.reshape(1, H), a0.reshape(1, 1), g0.reshape(1, H), be0.reshape(1, H),
      W1, b1.reshape(1, H), a1.reshape(1, 1), g1.reshape(1, H), be1.reshape(1, H),
      W2, b2.reshape(1, H), a2.reshape(1, 1), g2.reshape(1, H), be2.reshape(1, H),
      Wp, bp.reshape(1, 1))
    return out, w_out


# speedup vs baseline: 3.0272x; 2.9057x over previous
import math

import jax
import jax.numpy as jnp
from jax.experimental import pallas as pl
from jax.experimental.pallas import tpu as pltpu

N = 100000
D = 128
H = 128
NUM_GRAPHS = 512
BN_EPS = 1e-5
BLK = 4000
NBLK = N // BLK
WIN = 64
_BN_C = float(1.0 / math.sqrt(1.0 + BN_EPS))


def _fused_kernel(nf_ref, sm_ref, gid_ref, wa_ref, ba_ref,
                  W0_ref, b0_ref, a0_ref, g0_ref, be0_ref,
                  W1_ref, b1_ref, a1_ref, g1_ref, be1_ref,
                  W2_ref, b2_ref, a2_ref, g2_ref, be2_ref,
                  Wp_ref, bp_ref,
                  w_out_ref, out_ref, acc_ref):
    i = pl.program_id(0)
    nf = nf_ref[...]
    logit = jax.lax.dot_general(wa_ref[...], nf, (((1,), (1,)), ((), ())),
                                preferred_element_type=jnp.float32)
    w_row = jax.nn.sigmoid(logit + ba_ref[0, 0]) * sm_ref[0, 0, :][None, :]
    w_out_ref[...] = w_row.reshape(1, 1, BLK)
    gid = gid_ref[0, 0, :]

    @pl.when(i == 0)
    def _init():
        acc_ref[...] = jnp.zeros_like(acc_ref)

    first = gid_ref[0, 0, 0]
    last = gid_ref[0, 0, BLK - 1]
    row_iota = jax.lax.broadcasted_iota(jnp.int32, (WIN, BLK), 0)
    w_bcast = jnp.broadcast_to(w_row, (WIN, BLK))

    def _cond(carry):
        return carry <= last

    def _body(carry):
        base = jnp.minimum((carry // 8) * 8, NUM_GRAPHS - WIN)
        sel = (gid >= carry) & (gid < base + WIN)
        onehot = jnp.where(sel[None, :] & (row_iota == (gid - base)[None, :]),
                           w_bcast, 0.0)
        acc_ref[pl.ds(base, WIN), :] += jnp.dot(
            onehot, nf, preferred_element_type=jnp.float32)
        return base + WIN

    jax.lax.while_loop(_cond, _body, first)

    @pl.when(i == NBLK - 1)
    def _head():
        h = acc_ref[...]
        for W, b, a, g, be in ((W0_ref, b0_ref, a0_ref, g0_ref, be0_ref),
                               (W1_ref, b1_ref, a1_ref, g1_ref, be1_ref),
                               (W2_ref, b2_ref, a2_ref, g2_ref, be2_ref)):
            h = jnp.dot(h, W[...], precision=jax.lax.Precision.HIGHEST,
                        preferred_element_type=jnp.float32) + b[...]
            h = jnp.where(h >= 0, h, a[0, 0] * h)
            h = g[...] * (h * _BN_C) + be[...]
        out_ref[...] = jnp.dot(h, Wp_ref[...], precision=jax.lax.Precision.HIGHEST,
                               preferred_element_type=jnp.float32) + bp_ref[...]


@jax.jit
def kernel(node_feats, edge_feats, smask, graph_ids, w_atom, b_atom,
           W0, b0, a0, g0, be0, W1, b1, a1, g1, be1, W2, b2, a2, g2, be2,
           Wp, bp):
    del edge_feats
    sm3 = smask.reshape(NBLK, 1, BLK)
    gid3 = graph_ids.reshape(NBLK, 1, BLK)
    wa_row = w_atom.reshape(1, D)

    full = lambda *shape: pl.BlockSpec(shape, lambda i: (0,) * len(shape))
    in_specs = [
        pl.BlockSpec((BLK, D), lambda i: (i, 0)),
        pl.BlockSpec((1, 1, BLK), lambda i: (i, 0, 0)),
        pl.BlockSpec((1, 1, BLK), lambda i: (i, 0, 0)),
        full(1, D), full(1, 1),
    ]
    for _ in range(3):
        in_specs += [full(D, H), full(1, H), full(1, 1), full(1, H), full(1, H)]
    in_specs += [full(H, 1), full(1, 1)]

    w_out, out = pl.pallas_call(
        _fused_kernel,
        grid=(NBLK,),
        in_specs=in_specs,
        out_specs=[
            pl.BlockSpec((1, 1, BLK), lambda i: (i, 0, 0)),
            pl.BlockSpec((NUM_GRAPHS, 1), lambda i: (0, 0)),
        ],
        out_shape=[
            jax.ShapeDtypeStruct((NBLK, 1, BLK), jnp.float32),
            jax.ShapeDtypeStruct((NUM_GRAPHS, 1), jnp.float32),
        ],
        scratch_shapes=[pltpu.VMEM((NUM_GRAPHS, H), jnp.float32)],
    )(node_feats, sm3, gid3, wa_row, b_atom.reshape(1, 1),
      W0, b0.reshape(1, H), a0.reshape(1, 1), g0.reshape(1, H), be0.reshape(1, H),
      W1, b1.reshape(1, H), a1.reshape(1, 1), g1.reshape(1, H), be1.reshape(1, H),
      W2, b2.reshape(1, H), a2.reshape(1, 1), g2.reshape(1, H), be2.reshape(1, H),
      Wp, bp.reshape(1, 1))
    return out, w_out.reshape(N, 1)
